# baseline (speedup 1.0000x reference)
import functools

import jax
import jax.numpy as jnp
from jax import lax
from jax.experimental import pallas as pl
from jax.experimental.pallas import tpu as pltpu
from jax.experimental.pallas import tpu_sc as plsc

_NN, _NE, _NT, _F = 10000, 160000, 80000, 128
_G = 128
_NSUB, _NCORE = 16, 2
_NW = _NSUB * _NCORE
_ZR = 128
_NNP = 10240
_NTP = 80640
_NEP = 161280
_PREC = jax.lax.Precision.HIGHEST


def _mesh():
    return plsc.VectorSubcoreMesh(core_axis_name="c", subcore_axis_name="s",
                                  num_cores=_NCORE, num_subcores=_NSUB)


def _matmul_multi(h, ws, bm=4000):
    n = h.shape[0]
    nw = len(ws)

    def body(h_ref, *refs):
        hb = h_ref[...]
        for wr, orf in zip(refs[:nw], refs[nw:]):
            orf[...] = jnp.dot(hb, wr[...], preferred_element_type=jnp.float32,
                               precision=_PREC)

    return pl.pallas_call(
        body,
        grid=(n // bm,),
        in_specs=[pl.BlockSpec((bm, _F), lambda i: (i, 0))]
        + [pl.BlockSpec(w.shape, lambda i: (0, 0)) for w in ws],
        out_specs=[pl.BlockSpec((bm, w.shape[1]), lambda i: (i, 0)) for w in ws],
        out_shape=[jax.ShapeDtypeStruct((n, w.shape[1]), jnp.float32) for w in ws],
    )(h, *ws)


def _combine(d0, d2, h, w1):
    n = h.shape[0]
    bm = 4000

    def body(d0_ref, d2_ref, h_ref, w_ref, o_ref):
        acc = jnp.dot(h_ref[...], w_ref[...], preferred_element_type=jnp.float32,
                      precision=_PREC)
        o_ref[...] = jnp.maximum(acc + d0_ref[...] + d2_ref[...], 0.0)

    bs = pl.BlockSpec((bm, _F), lambda i: (i, 0))
    return pl.pallas_call(
        body,
        grid=(n // bm,),
        in_specs=[bs, bs, bs, pl.BlockSpec((_F, _F), lambda i: (0, 0))],
        out_specs=bs,
        out_shape=jax.ShapeDtypeStruct((n, _F), jnp.float32),
    )(d0, d2, h, w1)


def _addpair(a):
    _, n, f = a.shape
    bm = 2048

    def body(a_ref, b_ref, o_ref):
        o_ref[...] = a_ref[0] + b_ref[0]

    return pl.pallas_call(
        body,
        grid=(n // bm,),
        in_specs=[pl.BlockSpec((1, bm, f), lambda i: (0, i, 0)),
                  pl.BlockSpec((1, bm, f), lambda i: (1, i, 0))],
        out_specs=pl.BlockSpec((bm, f), lambda i: (i, 0)),
        out_shape=jax.ShapeDtypeStruct((n, f), jnp.float32),
    )(a, a)


def _zero_buf(buf, rows, f):
    zero16 = jnp.zeros((16,), jnp.float32)

    @pl.loop(0, rows)
    def _(r):
        for j in range(f // 16):
            buf[r, pl.ds(j * 16, 16)] = zero16


def _zero_acc_rows(zbuf, acc, r0, rw):
    for zi in range(rw // _ZR):
        pltpu.sync_copy(zbuf, acc.at[pl.ds(r0 + zi * _ZR, _ZR)])
    tail = rw % _ZR
    if tail:
        pltpu.sync_copy(zbuf.at[pl.ds(0, tail)],
                        acc.at[pl.ds(r0 + (rw // _ZR) * _ZR, tail)])


def _scale_rows(gbuf, val_v, f):
    @pl.loop(0, _G // 16)
    def _(cb):
        base = cb * 16
        v16 = val_v[pl.ds(base, 16)]
        for j in range(16):
            vv = jnp.full((16,), v16[j])
            for fc in range(f // 16):
                sl = (base + j, pl.ds(fc * 16, 16))
                gbuf[sl] = gbuf[sl] * vv


@functools.partial(jax.jit, static_argnames=("n_dst", "f"))
def _spmm_dup(dst, src, val, table, *, n_dst, f):
    nnz_pad = dst.shape[0]
    eps = nnz_pad // _NW
    ntiles = eps // _G
    rw = n_dst // _NSUB
    assert rw % 8 == 0

    @functools.partial(
        pl.kernel,
        out_type=jax.ShapeDtypeStruct((_NCORE, n_dst, f), jnp.float32),
        mesh=_mesh(),
        scratch_types=[
            pltpu.VMEM_SHARED((n_dst, f), jnp.float32),
            pltpu.VMEM((_G, f), jnp.float32),
            pltpu.VMEM((_G,), jnp.int32),
            pltpu.VMEM((_G,), jnp.int32),
            pltpu.VMEM((_G,), jnp.float32),
            pltpu.SemaphoreType.DMA,
        ],
    )
    def k(dst_hbm, src_hbm, val_hbm, tab_hbm, out_hbm,
          acc, gbuf, src_v, dst_v, val_v, sem):
        c = lax.axis_index("c")
        s = lax.axis_index("s")
        wid = s * _NCORE + c
        _zero_buf(gbuf, _G, f)
        _zero_acc_rows(gbuf, acc, s * rw, rw)
        plsc.subcore_barrier()

        base_e = wid * eps

        @pl.loop(0, ntiles)
        def _(t):
            e0 = base_e + t * _G
            pltpu.sync_copy(dst_hbm.at[pl.ds(e0, _G)], dst_v)
            pltpu.sync_copy(src_hbm.at[pl.ds(e0, _G)], src_v)
            pltpu.sync_copy(val_hbm.at[pl.ds(e0, _G)], val_v)
            pltpu.async_copy(tab_hbm.at[src_v], gbuf, sem).wait()
            _scale_rows(gbuf, val_v, f)
            pltpu.sync_copy(gbuf, acc.at[dst_v], add=True)

        plsc.subcore_barrier()
        pltpu.sync_copy(acc.at[pl.ds(s * rw, rw)],
                        out_hbm.at[c].at[pl.ds(s * rw, rw)])

    return k(dst, src, val, table)


@functools.partial(jax.jit, static_argnames=("n_dst", "seg_r"))
def _spmm_seg(dst, src, val, bounds, table, *, n_dst, seg_r):
    nseg = n_dst // seg_r
    spc = nseg // _NCORE
    rw = seg_r // _NSUB
    f = _F

    @functools.partial(
        pl.kernel,
        out_type=jax.ShapeDtypeStruct((n_dst, f), jnp.float32),
        mesh=_mesh(),
        scratch_types=[
            pltpu.VMEM_SHARED((seg_r + 1, f), jnp.float32),
            pltpu.VMEM((_G, f), jnp.float32),
            pltpu.VMEM((_G,), jnp.int32),
            pltpu.VMEM((_G,), jnp.int32),
            pltpu.VMEM((_G,), jnp.float32),
            pltpu.VMEM((16,), jnp.int32),
            pltpu.SemaphoreType.DMA,
        ],
    )
    def k(dst_hbm, src_hbm, val_hbm, bnd_hbm, tab_hbm, out_hbm,
          acc, gbuf, src_v, dst_v, val_v, bnd_v, sem):
        c = lax.axis_index("c")
        s = lax.axis_index("s")
        pltpu.sync_copy(bnd_hbm, bnd_v)
        iota16 = lax.iota(jnp.int32, 16)

        for p in range(spc):
            seg = 2 * p + c
            seg_base = seg * seg_r
            _zero_buf(gbuf, _G, f)
            _zero_acc_rows(gbuf, acc, s * rw, rw)
            plsc.subcore_barrier()

            bv = bnd_v[...]
            b0 = jnp.where(c == 0, bv[2 * p], bv[2 * p + 1])
            b1 = jnp.where(c == 0, bv[2 * p + 1], bv[2 * p + 2])
            ln = b1 - b0
            s0 = b0 + (ln * s) // _NSUB
            s1 = b0 + (ln * (s + 1)) // _NSUB
            st8 = (s0 // 8) * 8
            ntl = (s1 - st8 + _G - 1) // _G

            def tile_body(t, carry):
                e0 = st8 + t * _G
                pltpu.sync_copy(dst_hbm.at[pl.ds(e0, _G)], dst_v)
                pltpu.sync_copy(src_hbm.at[pl.ds(e0, _G)], src_v)
                pltpu.sync_copy(val_hbm.at[pl.ds(e0, _G)], val_v)
                pltpu.async_copy(tab_hbm.at[src_v], gbuf, sem).wait()
                for cb in range(_G // 16):
                    gidx = e0 + cb * 16 + iota16
                    m = (gidx >= s0) & (gidx < s1)
                    d16 = dst_v[pl.ds(cb * 16, 16)]
                    dst_v[pl.ds(cb * 16, 16)] = jnp.where(m, d16 - seg_base,
                                                          seg_r)
                    v16 = val_v[pl.ds(cb * 16, 16)]
                    val_v[pl.ds(cb * 16, 16)] = jnp.where(m, v16, 0.0)
                _scale_rows(gbuf, val_v, f)
                pltpu.sync_copy(gbuf, acc.at[dst_v], add=True)
                return carry

            lax.fori_loop(0, ntl, tile_body, 0)
            plsc.subcore_barrier()
            pltpu.sync_copy(acc.at[pl.ds(s * rw, rw)],
                            out_hbm.at[pl.ds(seg_base + s * rw, rw)])
            plsc.subcore_barrier()

    return k(dst, src, val, bounds, table)


def _pad_dup(dst, src, val, n_dst):
    n = dst.shape[0]
    m = _NW * _G
    npad = ((n + m - 1) // m) * m
    p = npad - n
    dst = jnp.concatenate([dst, jnp.full((p,), n_dst, jnp.int32)])
    src = jnp.concatenate([src, jnp.zeros((p,), jnp.int32)])
    val = jnp.concatenate([val, jnp.zeros((p,), jnp.float32)])
    return dst, src, val


def _sort_seg(dst, src, val, n_dst, seg_r):
    nseg = n_dst // seg_r
    order = jnp.argsort(dst)
    dst_s, src_s, val_s = dst[order], src[order], val[order]
    edges = jnp.arange(nseg + 1, dtype=jnp.int32) * seg_r
    bounds = jnp.searchsorted(dst_s, edges).astype(jnp.int32)
    bounds = jnp.concatenate(
        [bounds, jnp.zeros((16 - nseg - 1,), jnp.int32)])
    p = _G + 8
    dst_s = jnp.concatenate([dst_s, jnp.zeros((p,), jnp.int32)])
    src_s = jnp.concatenate([src_s, jnp.zeros((p,), jnp.int32)])
    val_s = jnp.concatenate([val_s, jnp.zeros((p,), jnp.float32)])
    return dst_s, src_s, val_s, bounds


def kernel(x, W0s, W1s, W2s, W0_L, B1_rows, B1_cols, B1_vals,
           B2_rows, B2_cols, B2_vals):
    b2c_d, b2c_s, b2c_v, b2c_b = _sort_seg(B2_cols, B2_rows, B2_vals,
                                           _NTP, 8064)
    b2r_d, b2r_s, b2r_v, b2r_b = _sort_seg(B2_rows, B2_cols, B2_vals,
                                           _NEP, 13440)
    b1c_d, b1c_s, b1c_v, b1c_b = _sort_seg(B1_cols, B1_rows, B1_vals,
                                           _NEP, 13440)
    b1d_d, b1d_s, b1d_v = _pad_dup(B1_rows, B1_cols, B1_vals, _NN)

    w0lp = jnp.concatenate(
        [W0_L.astype(jnp.float32), jnp.zeros((_F, _F - 1), jnp.float32)],
        axis=1)

    h = x
    for i in range(2):
        z0, z2 = _matmul_multi(h, [W0s[i], W2s[i]])
        t = _spmm_seg(b2c_d, b2c_s, b2c_v, b2c_b, z2, n_dst=_NTP, seg_r=8064)
        d2 = _spmm_seg(b2r_d, b2r_s, b2r_v, b2r_b, t, n_dst=_NEP, seg_r=13440)
        n2 = _spmm_dup(b1d_d, b1d_s, b1d_v, z0, n_dst=_NNP, f=_F)
        n = _addpair(n2)
        d0 = _spmm_seg(b1c_d, b1c_s, b1c_v, b1c_b, n, n_dst=_NEP, seg_r=13440)
        h = _combine(d0, d2, h, W1s[i])

    o2 = _spmm_dup(b1d_d, b1d_s, b1d_v, h, n_dst=_NNP, f=_F)
    o = _addpair(o2)
    (ofull,) = _matmul_multi(o, [w0lp], bm=2048)
    return ofull[:_NN, :1]


# speedup vs baseline: 1.0616x; 1.0616x over previous
import dataclasses
import functools

import jax
import jax.numpy as jnp
from jax import lax
from jax.experimental import pallas as pl
from jax.experimental.pallas import tpu as pltpu
from jax.experimental.pallas import tpu_sc as plsc

_NN, _NE, _NT, _F = 10000, 160000, 80000, 128
_G = 128
_NSUB, _NCORE = 16, 2
_NW = _NSUB * _NCORE
_ZR = 128
_NNP = 10240
_NTP = 80640
_NEP = 161280
_SEG_E = 11520
_SEG_T = 8064
_PREC = jax.lax.Precision.HIGHEST


_SC_CP = dataclasses.replace(pltpu.CompilerParams(),
                             needs_layout_passes=False)


def _mesh():
    return plsc.VectorSubcoreMesh(core_axis_name="c", subcore_axis_name="s",
                                  num_cores=_NCORE, num_subcores=_NSUB)


def _matmul_multi(h, ws, bm=4000):
    n = h.shape[0]
    nw = len(ws)

    def body(h_ref, *refs):
        hb = h_ref[...]
        for wr, orf in zip(refs[:nw], refs[nw:]):
            orf[...] = jnp.dot(hb, wr[...], preferred_element_type=jnp.float32,
                               precision=_PREC)

    return pl.pallas_call(
        body,
        grid=(n // bm,),
        in_specs=[pl.BlockSpec((bm, _F), lambda i: (i, 0))]
        + [pl.BlockSpec(w.shape, lambda i: (0, 0)) for w in ws],
        out_specs=[pl.BlockSpec((bm, w.shape[1]), lambda i: (i, 0)) for w in ws],
        out_shape=[jax.ShapeDtypeStruct((n, w.shape[1]), jnp.float32) for w in ws],
    )(h, *ws)


def _combine(d0, d2, h, w1):
    n = h.shape[0]
    bm = 4000

    def body(d0_ref, d2_ref, h_ref, w_ref, o_ref):
        acc = jnp.dot(h_ref[...], w_ref[...], preferred_element_type=jnp.float32,
                      precision=_PREC)
        o_ref[...] = jnp.maximum(acc + d0_ref[...] + d2_ref[...], 0.0)

    bs = pl.BlockSpec((bm, _F), lambda i: (i, 0))
    return pl.pallas_call(
        body,
        grid=(n // bm,),
        in_specs=[bs, bs, bs, pl.BlockSpec((_F, _F), lambda i: (0, 0))],
        out_specs=bs,
        out_shape=jax.ShapeDtypeStruct((n, _F), jnp.float32),
    )(d0, d2, h, w1)


def _addpair(a):
    _, n, f = a.shape
    bm = 2048

    def body(a_ref, b_ref, o_ref):
        o_ref[...] = a_ref[0] + b_ref[0]

    return pl.pallas_call(
        body,
        grid=(n // bm,),
        in_specs=[pl.BlockSpec((1, bm, f), lambda i: (0, i, 0)),
                  pl.BlockSpec((1, bm, f), lambda i: (1, i, 0))],
        out_specs=pl.BlockSpec((bm, f), lambda i: (i, 0)),
        out_shape=jax.ShapeDtypeStruct((n, f), jnp.float32),
    )(a, a)


def _zero_buf(buf, rows, f):
    zero16 = jnp.zeros((16,), jnp.float32)

    @pl.loop(0, rows)
    def _(r):
        for j in range(f // 16):
            buf[r, pl.ds(j * 16, 16)] = zero16


def _zero_acc_rows(zbuf, acc, r0, rw):
    for zi in range(rw // _ZR):
        pltpu.sync_copy(zbuf, acc.at[pl.ds(r0 + zi * _ZR, _ZR)])
    tail = rw % _ZR
    if tail:
        pltpu.sync_copy(zbuf.at[pl.ds(0, tail)],
                        acc.at[pl.ds(r0 + (rw // _ZR) * _ZR, tail)])


def _scale_rows(gbuf, val_v, f):
    @pl.loop(0, _G // 16)
    def _(cb):
        base = cb * 16
        v16 = val_v[pl.ds(base, 16)]
        for j in range(16):
            vv = jnp.full((16,), v16[j])
            for fc in range(f // 16):
                sl = (base + j, pl.ds(fc * 16, 16))
                gbuf[sl] = gbuf[sl] * vv


def _issue(cmb_hbm, tab_hbm, tidx, b3, gb, sem):
    pltpu.sync_copy(cmb_hbm.at[pl.ds(tidx * 3 * _G, 3 * _G)], b3)
    pltpu.async_copy(tab_hbm.at[b3.at[pl.ds(_G, _G)]], gb, sem)


def _wait_gather(tab_hbm, b3, gb, sem):
    pltpu.make_async_copy(tab_hbm.at[b3.at[pl.ds(_G, _G)]], gb, sem).wait()


def _extract16(bv, idx):
    sel = jnp.where(lax.iota(jnp.int32, 16) == idx, bv, 0)
    return jnp.sum(sel)


@functools.partial(jax.jit, static_argnames=("n_dst", "f"))
def _spmm_dup(cmb, table, *, n_dst, f):
    tps = (cmb.shape[0] // (3 * _G) - 1) // _NW
    nit = tps // 2
    rw = n_dst // _NSUB

    @functools.partial(
        pl.kernel,
        out_type=jax.ShapeDtypeStruct((_NCORE, n_dst, f), jnp.float32),
        mesh=_mesh(),
        compiler_params=_SC_CP,
        scratch_types=[
            pltpu.VMEM_SHARED((n_dst, f), jnp.float32),
            pltpu.VMEM((_G, f), jnp.float32),
            pltpu.VMEM((_G, f), jnp.float32),
            pltpu.VMEM((3 * _G,), jnp.int32),
            pltpu.VMEM((3 * _G,), jnp.int32),
            pltpu.VMEM((_G,), jnp.int32),
            pltpu.VMEM((_G,), jnp.int32),
            pltpu.VMEM((_G,), jnp.float32),
            pltpu.VMEM((_G,), jnp.float32),
            pltpu.SemaphoreType.DMA,
            pltpu.SemaphoreType.DMA,
        ],
    )
    def k(cmb_hbm, tab_hbm, out_hbm,
          acc, g0, g1, b30, b31, dl0, dl1, vf0, vf1, sem0, sem1):
        c = lax.axis_index("c")
        s = lax.axis_index("s")
        wid = s * _NCORE + c
        _zero_buf(g0, _G, f)
        _zero_acc_rows(g0, acc, s * rw, rw)
        plsc.subcore_barrier()

        t_base = wid * tps

        def process(gb, b3, dl, vf):
            for cb in range(_G // 16):
                sl = pl.ds(cb * 16, 16)
                dl[sl] = b3[sl]
                vf[sl] = plsc.bitcast(b3[pl.ds(2 * _G + cb * 16, 16)],
                                      jnp.float32)
            _scale_rows(gb, vf, f)
            pltpu.sync_copy(gb, acc.at[dl], add=True)

        _issue(cmb_hbm, tab_hbm, t_base, b30, g0, sem0)

        @pl.loop(0, nit)
        def _(i):
            t0 = t_base + 2 * i
            _issue(cmb_hbm, tab_hbm, t0 + 1, b31, g1, sem1)
            _wait_gather(tab_hbm, b30, g0, sem0)
            process(g0, b30, dl0, vf0)
            _issue(cmb_hbm, tab_hbm, t0 + 2, b30, g0, sem0)
            _wait_gather(tab_hbm, b31, g1, sem1)
            process(g1, b31, dl1, vf1)

        _wait_gather(tab_hbm, b30, g0, sem0)
        plsc.subcore_barrier()
        pltpu.sync_copy(acc.at[pl.ds(s * rw, rw)],
                        out_hbm.at[c].at[pl.ds(s * rw, rw)])

    return k(cmb, table)


@functools.partial(jax.jit, static_argnames=("n_dst", "seg_r"))
def _spmm_seg(cmb, bounds, table, *, n_dst, seg_r):
    nseg = n_dst // seg_r
    spc = nseg // _NCORE
    rw = seg_r // _NSUB
    f = _F

    @functools.partial(
        pl.kernel,
        out_type=jax.ShapeDtypeStruct((n_dst, f), jnp.float32),
        mesh=_mesh(),
        compiler_params=_SC_CP,
        scratch_types=[
            pltpu.VMEM_SHARED((seg_r + 1, f), jnp.float32),
            pltpu.VMEM((_G, f), jnp.float32),
            pltpu.VMEM((_G, f), jnp.float32),
            pltpu.VMEM((3 * _G,), jnp.int32),
            pltpu.VMEM((3 * _G,), jnp.int32),
            pltpu.VMEM((_G,), jnp.int32),
            pltpu.VMEM((_G,), jnp.int32),
            pltpu.VMEM((_G,), jnp.float32),
            pltpu.VMEM((_G,), jnp.float32),
            pltpu.VMEM((16,), jnp.int32),
            pltpu.SemaphoreType.DMA,
            pltpu.SemaphoreType.DMA,
        ],
    )
    def k(cmb_hbm, bnd_hbm, tab_hbm, out_hbm,
          acc, g0, g1, b30, b31, dl0, dl1, vf0, vf1, bnd_v, sem0, sem1):
        c = lax.axis_index("c")
        s = lax.axis_index("s")
        pltpu.sync_copy(bnd_hbm, bnd_v)
        iota16 = lax.iota(jnp.int32, 16)

        @pl.loop(0, spc)
        def _(p):
            seg = 2 * p + c
            seg_base = seg * seg_r
            _zero_buf(g0, _G, f)
            _zero_acc_rows(g0, acc, s * rw, rw)
            plsc.subcore_barrier()

            bv = bnd_v[...]
            b0 = _extract16(bv, seg)
            b1 = _extract16(bv, seg + 1)
            ln = b1 - b0
            s0 = b0 + (ln * s) // _NSUB
            s1 = b0 + (ln * (s + 1)) // _NSUB
            stg = (s0 // _G) * _G
            ntl = jnp.maximum((s1 - stg + _G - 1) // _G, 1)
            nit = (ntl + 1) // 2
            tb = stg // _G

            def process(gb, b3, dl, vf, e0):
                for cb in range(_G // 16):
                    sl = pl.ds(cb * 16, 16)
                    gidx = e0 + cb * 16 + iota16
                    m = (gidx >= s0) & (gidx < s1)
                    dl[sl] = jnp.where(m, b3[sl] - seg_base, seg_r)
                    v16 = plsc.bitcast(b3[pl.ds(2 * _G + cb * 16, 16)],
                                       jnp.float32)
                    vf[sl] = jnp.where(m, v16, 0.0)
                _scale_rows(gb, vf, f)
                pltpu.sync_copy(gb, acc.at[dl], add=True)

            _issue(cmb_hbm, tab_hbm, tb, b30, g0, sem0)

            def body(i, carry):
                t0 = tb + 2 * i
                e0 = stg + 2 * i * _G
                _issue(cmb_hbm, tab_hbm, t0 + 1, b31, g1, sem1)
                _wait_gather(tab_hbm, b30, g0, sem0)
                process(g0, b30, dl0, vf0, e0)
                _issue(cmb_hbm, tab_hbm, t0 + 2, b30, g0, sem0)
                _wait_gather(tab_hbm, b31, g1, sem1)
                process(g1, b31, dl1, vf1, e0 + _G)
                return carry

            lax.fori_loop(0, nit, body, 0)
            _wait_gather(tab_hbm, b30, g0, sem0)
            plsc.subcore_barrier()
            pltpu.sync_copy(acc.at[pl.ds(s * rw, rw)],
                            out_hbm.at[pl.ds(seg_base + s * rw, rw)])
            plsc.subcore_barrier()

    return k(cmb, bounds, table)


def _tile_pack(dst, src, val):
    t = dst.shape[0] // _G
    vi = lax.bitcast_convert_type(val, jnp.int32)
    cmb = jnp.stack([dst.reshape(t, _G), src.reshape(t, _G),
                     vi.reshape(t, _G)], axis=1)
    return cmb.reshape(-1)


def _pad_to(a, n, fill):
    return jnp.concatenate(
        [a, jnp.full((n - a.shape[0],), fill, a.dtype)])


def _prep_dup(dst, src, val):
    n = dst.shape[0]
    m = _NW * _G * 2
    npad = ((n + m - 1) // m) * m + _G
    return _tile_pack(_pad_to(dst, npad, 0), _pad_to(src, npad, 0),
                      _pad_to(val, npad, 0.0))


def _prep_seg(dst, src, val, n_dst, seg_r):
    nseg = n_dst // seg_r
    order = jnp.argsort(dst)
    dst_s, src_s, val_s = dst[order], src[order], val[order]
    marks = jnp.arange(nseg + 1, dtype=jnp.int32) * seg_r
    bounds = jnp.searchsorted(dst_s, marks).astype(jnp.int32)
    bounds = _pad_to(bounds, 16, 0)
    n = dst.shape[0]
    npad = ((n + _G - 1) // _G + 3) * _G
    cmb = _tile_pack(_pad_to(dst_s, npad, 0), _pad_to(src_s, npad, 0),
                     _pad_to(val_s, npad, 0.0))
    return cmb, bounds


def kernel(x, W0s, W1s, W2s, W0_L, B1_rows, B1_cols, B1_vals,
           B2_rows, B2_cols, B2_vals):
    b2c_cmb, b2c_b = _prep_seg(B2_cols, B2_rows, B2_vals, _NTP, _SEG_T)
    b2r_cmb, b2r_b = _prep_seg(B2_rows, B2_cols, B2_vals, _NEP, _SEG_E)
    b1c_cmb, b1c_b = _prep_seg(B1_cols, B1_rows, B1_vals, _NEP, _SEG_E)
    b1d_cmb = _prep_dup(B1_rows, B1_cols, B1_vals)

    w0lp = jnp.concatenate(
        [W0_L.astype(jnp.float32), jnp.zeros((_F, _F - 1), jnp.float32)],
        axis=1)

    h = x
    for i in range(2):
        z0, z2 = _matmul_multi(h, [W0s[i], W2s[i]])
        t = _spmm_seg(b2c_cmb, b2c_b, z2, n_dst=_NTP, seg_r=_SEG_T)
        d2 = _spmm_seg(b2r_cmb, b2r_b, t, n_dst=_NEP, seg_r=_SEG_E)
        n2 = _spmm_dup(b1d_cmb, z0, n_dst=_NNP, f=_F)
        n = _addpair(n2)
        d0 = _spmm_seg(b1c_cmb, b1c_b, n, n_dst=_NEP, seg_r=_SEG_E)
        h = _combine(d0, d2, h, W1s[i])

    o2 = _spmm_dup(b1d_cmb, h, n_dst=_NNP, f=_F)
    o = _addpair(o2)
    (ofull,) = _matmul_multi(o, [w0lp], bm=2048)
    return ofull[:_NN, :1]


# speedup vs baseline: 1.2248x; 1.1537x over previous
import dataclasses
import functools

import jax
import jax.numpy as jnp
from jax import lax
from jax.experimental import pallas as pl
from jax.experimental.pallas import tpu as pltpu
from jax.experimental.pallas import tpu_sc as plsc

_NN, _NE, _NT, _F = 10000, 160000, 80000, 128
_G = 128
_NSUB, _NCORE = 16, 2
_NW = _NSUB * _NCORE
_ZR = 128
_NNP = 10240
_NTP = 80640
_NEP = 161280
_SEG_E = 11520
_SEG_T = 8064
_SEG_N = 5120
_PREC = jax.lax.Precision.HIGHEST


_SC_CP = dataclasses.replace(pltpu.CompilerParams(),
                             needs_layout_passes=False)


def _mesh():
    return plsc.VectorSubcoreMesh(core_axis_name="c", subcore_axis_name="s",
                                  num_cores=_NCORE, num_subcores=_NSUB)


def _matmul_multi(h, ws, bm=4000):
    n = h.shape[0]
    nw = len(ws)

    def body(h_ref, *refs):
        hb = h_ref[...]
        for wr, orf in zip(refs[:nw], refs[nw:]):
            orf[...] = jnp.dot(hb, wr[...], preferred_element_type=jnp.float32,
                               precision=_PREC)

    return pl.pallas_call(
        body,
        grid=(n // bm,),
        in_specs=[pl.BlockSpec((bm, _F), lambda i: (i, 0))]
        + [pl.BlockSpec(w.shape, lambda i: (0, 0)) for w in ws],
        out_specs=[pl.BlockSpec((bm, w.shape[1]), lambda i: (i, 0)) for w in ws],
        out_shape=[jax.ShapeDtypeStruct((n, w.shape[1]), jnp.float32) for w in ws],
    )(h, *ws)


def _combine(d0, d2, h, w1):
    n = h.shape[0]
    bm = 4000

    def body(d0_ref, d2_ref, h_ref, w_ref, o_ref):
        acc = jnp.dot(h_ref[...], w_ref[...], preferred_element_type=jnp.float32,
                      precision=_PREC)
        o_ref[...] = jnp.maximum(acc + d0_ref[...] + d2_ref[...], 0.0)

    bs = pl.BlockSpec((bm, _F), lambda i: (i, 0))
    return pl.pallas_call(
        body,
        grid=(n // bm,),
        in_specs=[bs, bs, bs, pl.BlockSpec((_F, _F), lambda i: (0, 0))],
        out_specs=bs,
        out_shape=jax.ShapeDtypeStruct((n, _F), jnp.float32),
    )(d0, d2, h, w1)


def _zero_buf(buf, rows, f):
    zero16 = jnp.zeros((16,), jnp.float32)

    @pl.loop(0, rows)
    def _(r):
        for j in range(f // 16):
            buf[r, pl.ds(j * 16, 16)] = zero16


def _zero_acc_rows(zbuf, acc, r0, rw):
    for zi in range(rw // _ZR):
        pltpu.sync_copy(zbuf, acc.at[pl.ds(r0 + zi * _ZR, _ZR)])
    tail = rw % _ZR
    if tail:
        pltpu.sync_copy(zbuf.at[pl.ds(0, tail)],
                        acc.at[pl.ds(r0 + (rw // _ZR) * _ZR, tail)])


def _scale_rows(gbuf, val_v, f):
    @pl.loop(0, _G // 16)
    def _(cb):
        base = cb * 16
        v16 = val_v[pl.ds(base, 16)]
        for j in range(16):
            vv = jnp.full((16,), v16[j])
            for fc in range(f // 16):
                sl = (base + j, pl.ds(fc * 16, 16))
                gbuf[sl] = gbuf[sl] * vv


def _issue(cmb_hbm, tab_hbm, tidx, b3, gb, sem):
    pltpu.sync_copy(cmb_hbm.at[pl.ds(tidx * 3 * _G, 3 * _G)], b3)
    pltpu.async_copy(tab_hbm.at[b3.at[pl.ds(_G, _G)]], gb, sem)


def _wait_gather(tab_hbm, b3, gb, sem):
    pltpu.make_async_copy(tab_hbm.at[b3.at[pl.ds(_G, _G)]], gb, sem).wait()


def _extract16(bv, idx):
    sel = jnp.where(lax.iota(jnp.int32, 16) == idx, bv, 0)
    return jnp.sum(sel)


@functools.partial(jax.jit, static_argnames=("n_dst", "seg_r"))
def _spmm_seg(cmb, bounds, table, *, n_dst, seg_r):
    nseg = n_dst // seg_r
    spc = nseg // _NCORE
    rw = seg_r // _NSUB
    f = _F

    @functools.partial(
        pl.kernel,
        out_type=jax.ShapeDtypeStruct((n_dst, f), jnp.float32),
        mesh=_mesh(),
        compiler_params=_SC_CP,
        scratch_types=[
            pltpu.VMEM_SHARED((seg_r + 1, f), jnp.float32),
            pltpu.VMEM((_G, f), jnp.float32),
            pltpu.VMEM((_G, f), jnp.float32),
            pltpu.VMEM((3 * _G,), jnp.int32),
            pltpu.VMEM((3 * _G,), jnp.int32),
            pltpu.VMEM((_G,), jnp.int32),
            pltpu.VMEM((_G,), jnp.int32),
            pltpu.VMEM((_G,), jnp.float32),
            pltpu.VMEM((_G,), jnp.float32),
            pltpu.VMEM((16,), jnp.int32),
            pltpu.SemaphoreType.DMA,
            pltpu.SemaphoreType.DMA,
        ],
    )
    def k(cmb_hbm, bnd_hbm, tab_hbm, out_hbm,
          acc, g0, g1, b30, b31, dl0, dl1, vf0, vf1, bnd_v, sem0, sem1):
        c = lax.axis_index("c")
        s = lax.axis_index("s")
        pltpu.sync_copy(bnd_hbm, bnd_v)
        iota16 = lax.iota(jnp.int32, 16)

        @pl.loop(0, spc)
        def _(p):
            seg = 2 * p + c
            seg_base = seg * seg_r
            _zero_buf(g0, _G, f)
            _zero_acc_rows(g0, acc, s * rw, rw)
            plsc.subcore_barrier()

            bv = bnd_v[...]
            b0 = _extract16(bv, seg)
            b1 = _extract16(bv, seg + 1)
            ln = b1 - b0
            s0 = b0 + (ln * s) // _NSUB
            s1 = b0 + (ln * (s + 1)) // _NSUB
            stg = (s0 // _G) * _G
            ntl = jnp.maximum((s1 - stg + _G - 1) // _G, 1)
            nit = (ntl + 1) // 2
            tb = stg // _G

            def process(gb, b3, dl, vf, e0):
                for cb in range(_G // 16):
                    sl = pl.ds(cb * 16, 16)
                    gidx = e0 + cb * 16 + iota16
                    m = (gidx >= s0) & (gidx < s1)
                    dl[sl] = jnp.where(m, b3[sl] - seg_base, seg_r)
                    v16 = plsc.bitcast(b3[pl.ds(2 * _G + cb * 16, 16)],
                                       jnp.float32)
                    vf[sl] = jnp.where(m, v16, 0.0)
                _scale_rows(gb, vf, f)
                pltpu.sync_copy(gb, acc.at[dl], add=True)

            _issue(cmb_hbm, tab_hbm, tb, b30, g0, sem0)

            def body(i, carry):
                t0 = tb + 2 * i
                e0 = stg + 2 * i * _G
                _issue(cmb_hbm, tab_hbm, t0 + 1, b31, g1, sem1)
                _wait_gather(tab_hbm, b30, g0, sem0)
                process(g0, b30, dl0, vf0, e0)
                _issue(cmb_hbm, tab_hbm, t0 + 2, b30, g0, sem0)
                _wait_gather(tab_hbm, b31, g1, sem1)
                process(g1, b31, dl1, vf1, e0 + _G)
                return carry

            lax.fori_loop(0, nit, body, 0)
            _wait_gather(tab_hbm, b30, g0, sem0)
            plsc.subcore_barrier()
            pltpu.sync_copy(acc.at[pl.ds(s * rw, rw)],
                            out_hbm.at[pl.ds(seg_base + s * rw, rw)])
            plsc.subcore_barrier()

    return k(cmb, bounds, table)


def _tile_pack(dst, src, val):
    t = dst.shape[0] // _G
    vi = lax.bitcast_convert_type(val, jnp.int32)
    cmb = jnp.stack([dst.reshape(t, _G), src.reshape(t, _G),
                     vi.reshape(t, _G)], axis=1)
    return cmb.reshape(-1)


def _pad_to(a, n, fill):
    return jnp.concatenate(
        [a, jnp.full((n - a.shape[0],), fill, a.dtype)])


def _prep_seg(dst, src, val, n_dst, seg_r):
    nseg = n_dst // seg_r
    dst_s, src_s, val_s = lax.sort((dst, src, val), num_keys=1)
    marks = jnp.arange(nseg + 1, dtype=jnp.int32) * seg_r
    bounds = jnp.searchsorted(dst_s, marks).astype(jnp.int32)
    bounds = _pad_to(bounds, 16, 0)
    n = dst.shape[0]
    npad = ((n + _G - 1) // _G + 3) * _G
    cmb = _tile_pack(_pad_to(dst_s, npad, 0), _pad_to(src_s, npad, 0),
                     _pad_to(val_s, npad, 0.0))
    return cmb, bounds


def kernel(x, W0s, W1s, W2s, W0_L, B1_rows, B1_cols, B1_vals,
           B2_rows, B2_cols, B2_vals):
    b2c_cmb, b2c_b = _prep_seg(B2_cols, B2_rows, B2_vals, _NTP, _SEG_T)
    b2r_cmb, b2r_b = _prep_seg(B2_rows, B2_cols, B2_vals, _NEP, _SEG_E)
    b1c_cmb, b1c_b = _prep_seg(B1_cols, B1_rows, B1_vals, _NEP, _SEG_E)
    b1r_cmb, b1r_b = _prep_seg(B1_rows, B1_cols, B1_vals, _NNP, _SEG_N)

    w0lp = jnp.concatenate(
        [W0_L.astype(jnp.float32), jnp.zeros((_F, _F - 1), jnp.float32)],
        axis=1)

    h = x
    for i in range(2):
        z0, z2 = _matmul_multi(h, [W0s[i], W2s[i]])
        t = _spmm_seg(b2c_cmb, b2c_b, z2, n_dst=_NTP, seg_r=_SEG_T)
        d2 = _spmm_seg(b2r_cmb, b2r_b, t, n_dst=_NEP, seg_r=_SEG_E)
        n = _spmm_seg(b1r_cmb, b1r_b, z0, n_dst=_NNP, seg_r=_SEG_N)
        d0 = _spmm_seg(b1c_cmb, b1c_b, n, n_dst=_NEP, seg_r=_SEG_E)
        h = _combine(d0, d2, h, W1s[i])

    o = _spmm_seg(b1r_cmb, b1r_b, h, n_dst=_NNP, seg_r=_SEG_N)
    (ofull,) = _matmul_multi(o, [w0lp], bm=2048)
    return ofull[:_NN, :1]


# speedup vs baseline: 1.2710x; 1.0378x over previous
import dataclasses
import functools

import jax
import jax.numpy as jnp
from jax import lax
from jax.experimental import pallas as pl
from jax.experimental.pallas import tpu as pltpu
from jax.experimental.pallas import tpu_sc as plsc

_NN, _NE, _NT, _F = 10000, 160000, 80000, 128
_G = 128
_NSUB, _NCORE = 16, 2
_NW = _NSUB * _NCORE
_ZR = 128
_NNP = 10240
_NTP = 80640
_NEP = 161280
_SEG_E = 11520
_SEG_T = 8064
_SEG_N = 5120
_PREC = jax.lax.Precision.DEFAULT


_SC_CP = dataclasses.replace(pltpu.CompilerParams(),
                             needs_layout_passes=False)


def _mesh():
    return plsc.VectorSubcoreMesh(core_axis_name="c", subcore_axis_name="s",
                                  num_cores=_NCORE, num_subcores=_NSUB)


def _matmul_multi(h, ws, bm=4000):
    n = h.shape[0]
    nw = len(ws)

    def body(h_ref, *refs):
        hb = h_ref[...]
        for wr, orf in zip(refs[:nw], refs[nw:]):
            orf[...] = jnp.dot(hb, wr[...], preferred_element_type=jnp.float32,
                               precision=_PREC)

    return pl.pallas_call(
        body,
        grid=(n // bm,),
        in_specs=[pl.BlockSpec((bm, _F), lambda i: (i, 0))]
        + [pl.BlockSpec(w.shape, lambda i: (0, 0)) for w in ws],
        out_specs=[pl.BlockSpec((bm, w.shape[1]), lambda i: (i, 0)) for w in ws],
        out_shape=[jax.ShapeDtypeStruct((n, w.shape[1]), jnp.float32) for w in ws],
    )(h, *ws)


def _combine(d0, d2, h, w1):
    n = h.shape[0]
    bm = 4000

    def body(d0_ref, d2_ref, h_ref, w_ref, o_ref):
        acc = jnp.dot(h_ref[...], w_ref[...], preferred_element_type=jnp.float32,
                      precision=_PREC)
        o_ref[...] = jnp.maximum(acc + d0_ref[...] + d2_ref[...], 0.0)

    bs = pl.BlockSpec((bm, _F), lambda i: (i, 0))
    return pl.pallas_call(
        body,
        grid=(n // bm,),
        in_specs=[bs, bs, bs, pl.BlockSpec((_F, _F), lambda i: (0, 0))],
        out_specs=bs,
        out_shape=jax.ShapeDtypeStruct((n, _F), jnp.float32),
    )(d0, d2, h, w1)


def _zero_buf(buf, rows, f):
    zero16 = jnp.zeros((16,), jnp.float32)

    @pl.loop(0, rows)
    def _(r):
        for j in range(f // 16):
            buf[r, pl.ds(j * 16, 16)] = zero16


def _zero_acc_rows(zbuf, acc, r0, rw):
    for zi in range(rw // _ZR):
        pltpu.sync_copy(zbuf, acc.at[pl.ds(r0 + zi * _ZR, _ZR)])
    tail = rw % _ZR
    if tail:
        pltpu.sync_copy(zbuf.at[pl.ds(0, tail)],
                        acc.at[pl.ds(r0 + (rw // _ZR) * _ZR, tail)])


def _scale_rows(gbuf, val_v, f):
    @pl.loop(0, _G // 16)
    def _(cb):
        base = cb * 16
        v16 = val_v[pl.ds(base, 16)]
        for j in range(16):
            vv = jnp.full((16,), v16[j])
            for fc in range(f // 16):
                sl = (base + j, pl.ds(fc * 16, 16))
                gbuf[sl] = gbuf[sl] * vv


def _issue(cmb_hbm, tab_hbm, tidx, b3, gb, sem):
    pltpu.sync_copy(cmb_hbm.at[pl.ds(tidx * 3 * _G, 3 * _G)], b3)
    pltpu.async_copy(tab_hbm.at[b3.at[pl.ds(_G, _G)]], gb, sem)


def _wait_gather(tab_hbm, b3, gb, sem):
    pltpu.make_async_copy(tab_hbm.at[b3.at[pl.ds(_G, _G)]], gb, sem).wait()


def _extract16(bv, idx):
    sel = jnp.where(lax.iota(jnp.int32, 16) == idx, bv, 0)
    return jnp.sum(sel)


@functools.partial(jax.jit, static_argnames=("n_dst", "seg_r"))
def _spmm_seg(cmb, bounds, table, *, n_dst, seg_r):
    nseg = n_dst // seg_r
    spc = nseg // _NCORE
    rw = seg_r // _NSUB
    f = _F

    @functools.partial(
        pl.kernel,
        out_type=jax.ShapeDtypeStruct((n_dst, f), jnp.float32),
        mesh=_mesh(),
        compiler_params=_SC_CP,
        scratch_types=[
            pltpu.VMEM_SHARED((seg_r + 1, f), jnp.float32),
            pltpu.VMEM((_G, f), jnp.float32),
            pltpu.VMEM((_G, f), jnp.float32),
            pltpu.VMEM((3 * _G,), jnp.int32),
            pltpu.VMEM((3 * _G,), jnp.int32),
            pltpu.VMEM((_G,), jnp.int32),
            pltpu.VMEM((_G,), jnp.int32),
            pltpu.VMEM((_G,), jnp.float32),
            pltpu.VMEM((_G,), jnp.float32),
            pltpu.VMEM((16,), jnp.int32),
            pltpu.SemaphoreType.DMA,
            pltpu.SemaphoreType.DMA,
        ],
    )
    def k(cmb_hbm, bnd_hbm, tab_hbm, out_hbm,
          acc, g0, g1, b30, b31, dl0, dl1, vf0, vf1, bnd_v, sem0, sem1):
        c = lax.axis_index("c")
        s = lax.axis_index("s")
        pltpu.sync_copy(bnd_hbm, bnd_v)
        iota16 = lax.iota(jnp.int32, 16)

        @pl.loop(0, spc)
        def _(p):
            seg = 2 * p + c
            seg_base = seg * seg_r
            _zero_buf(g0, _G, f)
            _zero_acc_rows(g0, acc, s * rw, rw)
            plsc.subcore_barrier()

            bv = bnd_v[...]
            b0 = _extract16(bv, seg)
            b1 = _extract16(bv, seg + 1)
            ln = b1 - b0
            s0 = b0 + (ln * s) // _NSUB
            s1 = b0 + (ln * (s + 1)) // _NSUB
            stg = (s0 // _G) * _G
            ntl = jnp.maximum((s1 - stg + _G - 1) // _G, 1)
            nit = (ntl + 1) // 2
            tb = stg // _G

            def process(gb, b3, dl, vf, e0):
                for cb in range(_G // 16):
                    sl = pl.ds(cb * 16, 16)
                    gidx = e0 + cb * 16 + iota16
                    m = (gidx >= s0) & (gidx < s1)
                    dl[sl] = jnp.where(m, b3[sl] - seg_base, seg_r)
                    v16 = plsc.bitcast(b3[pl.ds(2 * _G + cb * 16, 16)],
                                       jnp.float32)
                    vf[sl] = jnp.where(m, v16, 0.0)
                _scale_rows(gb, vf, f)
                pltpu.sync_copy(gb, acc.at[dl], add=True)

            _issue(cmb_hbm, tab_hbm, tb, b30, g0, sem0)

            def body(i, carry):
                t0 = tb + 2 * i
                e0 = stg + 2 * i * _G
                _issue(cmb_hbm, tab_hbm, t0 + 1, b31, g1, sem1)
                _wait_gather(tab_hbm, b30, g0, sem0)
                process(g0, b30, dl0, vf0, e0)
                _issue(cmb_hbm, tab_hbm, t0 + 2, b30, g0, sem0)
                _wait_gather(tab_hbm, b31, g1, sem1)
                process(g1, b31, dl1, vf1, e0 + _G)
                return carry

            lax.fori_loop(0, nit, body, 0)
            _wait_gather(tab_hbm, b30, g0, sem0)
            plsc.subcore_barrier()
            pltpu.sync_copy(acc.at[pl.ds(s * rw, rw)],
                            out_hbm.at[pl.ds(seg_base + s * rw, rw)])
            plsc.subcore_barrier()

    return k(cmb, bounds, table)


def _tile_pack(dst, src, val):
    t = dst.shape[0] // _G
    vi = lax.bitcast_convert_type(val, jnp.int32)
    cmb = jnp.stack([dst.reshape(t, _G), src.reshape(t, _G),
                     vi.reshape(t, _G)], axis=1)
    return cmb.reshape(-1)


def _pad_to(a, n, fill):
    return jnp.concatenate(
        [a, jnp.full((n - a.shape[0],), fill, a.dtype)])


def _prep_seg(dst, src, val, n_dst, seg_r):
    nseg = n_dst // seg_r
    dst_s, src_s, val_s = lax.sort((dst, src, val), num_keys=1)
    marks = jnp.arange(nseg + 1, dtype=jnp.int32) * seg_r
    bounds = jnp.searchsorted(dst_s, marks).astype(jnp.int32)
    bounds = _pad_to(bounds, 16, 0)
    n = dst.shape[0]
    npad = ((n + _G - 1) // _G + 3) * _G
    cmb = _tile_pack(_pad_to(dst_s, npad, 0), _pad_to(src_s, npad, 0),
                     _pad_to(val_s, npad, 0.0))
    return cmb, bounds


def kernel(x, W0s, W1s, W2s, W0_L, B1_rows, B1_cols, B1_vals,
           B2_rows, B2_cols, B2_vals):
    b2c_cmb, b2c_b = _prep_seg(B2_cols, B2_rows, B2_vals, _NTP, _SEG_T)
    b2r_cmb, b2r_b = _prep_seg(B2_rows, B2_cols, B2_vals, _NEP, _SEG_E)
    b1c_cmb, b1c_b = _prep_seg(B1_cols, B1_rows, B1_vals, _NEP, _SEG_E)
    b1r_cmb, b1r_b = _prep_seg(B1_rows, B1_cols, B1_vals, _NNP, _SEG_N)

    w0lp = jnp.concatenate(
        [W0_L.astype(jnp.float32), jnp.zeros((_F, _F - 1), jnp.float32)],
        axis=1)

    h = x
    for i in range(2):
        z0, z2 = _matmul_multi(h, [W0s[i], W2s[i]])
        t = _spmm_seg(b2c_cmb, b2c_b, z2, n_dst=_NTP, seg_r=_SEG_T)
        d2 = _spmm_seg(b2r_cmb, b2r_b, t, n_dst=_NEP, seg_r=_SEG_E)
        n = _spmm_seg(b1r_cmb, b1r_b, z0, n_dst=_NNP, seg_r=_SEG_N)
        d0 = _spmm_seg(b1c_cmb, b1c_b, n, n_dst=_NEP, seg_r=_SEG_E)
        h = _combine(d0, d2, h, W1s[i])

    o = _spmm_seg(b1r_cmb, b1r_b, h, n_dst=_NNP, seg_r=_SEG_N)
    (ofull,) = _matmul_multi(o, [w0lp], bm=2048)
    return ofull[:_NN, :1]
